# baseline (speedup 1.0000x reference)
import functools

import jax
import jax.numpy as jnp
from jax import lax
from jax.experimental import pallas as pl
from jax.experimental.pallas import tpu as pltpu
from jax.experimental.pallas import tpu_sc as plsc

NC = 2
NS = 16
L = 16
NW = NC * NS


def _build(B, N, R, nc=NC):
    nw = nc * NS
    WPB = nw // B
    ROWS = N // WPB
    NJ = N // L

    mesh = plsc.VectorSubcoreMesh(
        core_axis_name="c", subcore_axis_name="s",
        num_cores=nc, num_subcores=NS)

    @functools.partial(
        pl.kernel,
        out_type=jax.ShapeDtypeStruct((nw, L), jnp.float32),
        mesh=mesh,
        compiler_params=pltpu.CompilerParams(needs_layout_passes=False),
        scratch_types=[
            pltpu.VMEM((3, N), jnp.float32),
            pltpu.VMEM((3, N), jnp.float32),
            pltpu.VMEM((N,), jnp.float32),
            pltpu.VMEM((N,), jnp.float32),
            pltpu.VMEM((L,), jnp.float32),
            pltpu.VMEM((L * L,), jnp.float32),
        ],
    )
    def chamfer(pred_hbm, gt_hbm, out_hbm, p_v, g_v, pn_v, gn_v, acc_v,
                tr_v):
        cid = lax.axis_index("c")
        sid = lax.axis_index("s")
        wid = cid * NS + sid
        b = wid // WPB
        r0 = (wid % WPB) * ROWS

        pltpu.sync_copy(pred_hbm.at[b], p_v)
        pltpu.sync_copy(gt_hbm.at[b], g_v)

        def norms(j, _):
            sl = pl.ds(j * L, L)
            px, py, pz = p_v[0, sl], p_v[1, sl], p_v[2, sl]
            pn_v[sl] = px * px + py * py + pz * pz
            gx, gy, gz = g_v[0, sl], g_v[1, sl], g_v[2, sl]
            gn_v[sl] = gx * gx + gy * gy + gz * gz
            return 0
        lax.fori_loop(0, NJ, norms, 0, unroll=False)

        def direction(src_v, srcn_v, tgt_v, tgtn_v, acc):
            lanes = jnp.arange(L, dtype=jnp.int32)

            def outer(i, acc):
                n0 = r0 + i * L
                sl0 = pl.ds(n0, L)
                rxv = -2.0 * src_v[0, sl0]
                ryv = -2.0 * src_v[1, sl0]
                rzv = -2.0 * src_v[2, sl0]
                snv = srcn_v[sl0]
                all_rmins = []
                for g in range(L // R):
                    sx = [rxv[g * R + r] for r in range(R)]
                    sy = [ryv[g * R + r] for r in range(R)]
                    sz = [rzv[g * R + r] for r in range(R)]

                    def inner(j, rmins):
                        sl = pl.ds(j * L, L)
                        tx, ty, tz = tgt_v[0, sl], tgt_v[1, sl], tgt_v[2, sl]
                        tn = tgtn_v[sl]
                        return tuple(
                            jnp.minimum(
                                rmins[r],
                                tn + sx[r] * tx + sy[r] * ty + sz[r] * tz)
                            for r in range(R))

                    init = tuple(jnp.full((L,), jnp.inf, jnp.float32)
                                 for _ in range(R))
                    all_rmins.extend(
                        lax.fori_loop(0, NJ, inner, init, unroll=False))
                for r in range(L):
                    plsc.store_scatter(tr_v, [lanes * L + r], all_rmins[r])
                rowmin = tr_v[pl.ds(0, L)]
                for l in range(1, L):
                    rowmin = jnp.minimum(rowmin, tr_v[pl.ds(l * L, L)])
                return acc + jnp.maximum(snv + rowmin, 0.0)
            return lax.fori_loop(0, ROWS // L, outer, acc, unroll=False)

        acc = jnp.zeros((L,), jnp.float32)
        acc = direction(p_v, pn_v, g_v, gn_v, acc)
        acc = direction(g_v, gn_v, p_v, pn_v, acc)
        acc_v[...] = acc
        pltpu.sync_copy(acc_v, out_hbm.at[wid])

    return chamfer


def _build_tc(B, N, BN):
    NBLK = N // BN

    def body(p_ref, g_ref, out_ref):
        b = pl.program_id(0)
        pblk = p_ref[0]
        gblk = g_ref[0]
        pn = jnp.sum(pblk * pblk, axis=1, keepdims=True)
        gx, gy, gz = gblk[0:1, :], gblk[1:2, :], gblk[2:3, :]
        gn = 0.25 * (gx * gx + gy * gy + gz * gz)
        g16 = gblk.astype(jnp.bfloat16)
        colmin = jnp.full((1, N), jnp.inf, jnp.float32)
        for i in range(NBLK):
            lo, hi = i * BN, (i + 1) * BN
            cprime = jnp.dot(pblk[lo:hi, :].astype(jnp.bfloat16), g16,
                             preferred_element_type=jnp.float32)
            d2 = (pn[lo:hi, :] + gn) + cprime
            xs = jnp.min(d2, axis=1, keepdims=True)
            out_x = jnp.sum(jnp.maximum(xs, 0.0))
            prev = jnp.where(i == 0, 0.0, out_ref[b, 0])
            out_ref[b, 0] = prev + out_x
            colmin = jnp.minimum(colmin, jnp.min(d2, axis=0, keepdims=True))
        out_ref[b, 1] = jnp.sum(jnp.maximum(colmin, 0.0))

    return pl.pallas_call(
        body,
        grid=(B,),
        in_specs=[
            pl.BlockSpec((1, N, 3), lambda b: (b, 0, 0)),
            pl.BlockSpec((1, 3, N), lambda b: (b, 0, 0)),
        ],
        out_specs=pl.BlockSpec((B, 2), lambda b: (0, 0),
                               memory_space=pltpu.SMEM),
        out_shape=jax.ShapeDtypeStruct((B, 2), jnp.float32),
    )


_N = 2048
_B_SC = 0
_B_TC = 8 - _B_SC
_chamfer_sc = _build(_B_SC, _N, 8) if _B_SC else None
_chamfer_tc = _build_tc(_B_TC, _N, 2048)


def kernel(pred_points, gt_points):
    B, N, _ = pred_points.shape
    pred = pred_points.astype(jnp.float32)
    gt = gt_points.astype(jnp.float32)

    g2T = jnp.swapaxes(-2.0 * gt[:_B_TC], 1, 2)
    tc_parts = _chamfer_tc(pred[:_B_TC], g2T)
    total = jnp.sum(tc_parts)

    if _B_SC:
        pred_t = jnp.swapaxes(pred[_B_TC:], 1, 2)
        gt_t = jnp.swapaxes(gt[_B_TC:], 1, 2)
        sc_parts = _chamfer_sc(pred_t, gt_t)
        total = total + jnp.sum(sc_parts)

    return total / (B * N)


# speedup vs baseline: 1.0663x; 1.0663x over previous
import jax
import jax.numpy as jnp
from jax.experimental import pallas as pl
from jax.experimental.pallas import tpu as pltpu


def _build_tc(B, N, BN):
    NBLK = N // BN

    def body(p_ref, g_ref, out_ref):
        b = pl.program_id(0)
        pblk = p_ref[0]
        gblk = g_ref[0]
        pn = jnp.sum(pblk * pblk, axis=1, keepdims=True)
        gx, gy, gz = gblk[0:1, :], gblk[1:2, :], gblk[2:3, :]
        gn = 0.25 * (gx * gx + gy * gy + gz * gz)
        g16 = gblk.astype(jnp.bfloat16)
        colmin = jnp.full((1, N), jnp.inf, jnp.float32)
        for i in range(NBLK):
            lo, hi = i * BN, (i + 1) * BN
            cprime = jnp.dot(pblk[lo:hi, :].astype(jnp.bfloat16), g16,
                             preferred_element_type=jnp.float32)
            d2 = (pn[lo:hi, :] + gn) + cprime
            xs = jnp.min(d2, axis=1, keepdims=True)
            out_x = jnp.sum(jnp.maximum(xs, 0.0))
            prev = jnp.where(i == 0, 0.0, out_ref[b, 0])
            out_ref[b, 0] = prev + out_x
            colmin = jnp.minimum(colmin, jnp.min(d2, axis=0, keepdims=True))
        out_ref[b, 1] = jnp.sum(jnp.maximum(colmin, 0.0))

    return pl.pallas_call(
        body,
        grid=(B,),
        in_specs=[
            pl.BlockSpec((1, N, 3), lambda b: (b, 0, 0)),
            pl.BlockSpec((1, 3, N), lambda b: (b, 0, 0)),
        ],
        out_specs=pl.BlockSpec((B, 2), lambda b: (0, 0),
                               memory_space=pltpu.SMEM),
        out_shape=jax.ShapeDtypeStruct((B, 2), jnp.float32),
    )


_chamfer_tc = _build_tc(8, 2048, 1024)


def kernel(pred_points, gt_points):
    B, N, _ = pred_points.shape
    pred = pred_points.astype(jnp.float32)
    gt = gt_points.astype(jnp.float32)
    g2T = jnp.swapaxes(-2.0 * gt, 1, 2)
    parts = _chamfer_tc(pred, g2T)
    return jnp.sum(parts) / (B * N)


# speedup vs baseline: 1.0846x; 1.0172x over previous
import jax
import jax.numpy as jnp
from jax.experimental import pallas as pl
from jax.experimental.pallas import tpu as pltpu


def _build_tc(B, N, BN, BPB):
    NBLK = N // BN
    GB = B // BPB

    def body(p_ref, g_ref, out_ref):
        gb = pl.program_id(0)
        for j in range(BPB):
            pblk = p_ref[j]
            gblk = g_ref[j]
            pn = jnp.sum(pblk * pblk, axis=1, keepdims=True)
            gx, gy, gz = gblk[0:1, :], gblk[1:2, :], gblk[2:3, :]
            gn = 0.25 * (gx * gx + gy * gy + gz * gz)
            g16 = gblk.astype(jnp.bfloat16)
            colmin = jnp.full((1, N), jnp.inf, jnp.float32)
            xsum = jnp.float32(0.0)
            for i in range(NBLK):
                lo, hi = i * BN, (i + 1) * BN
                cprime = jnp.dot(pblk[lo:hi, :].astype(jnp.bfloat16), g16,
                                 preferred_element_type=jnp.float32)
                d2 = (pn[lo:hi, :] + gn) + cprime
                xs = jnp.min(d2, axis=1, keepdims=True)
                xsum = xsum + jnp.sum(jnp.maximum(xs, 0.0))
                colmin = jnp.minimum(colmin,
                                     jnp.min(d2, axis=0, keepdims=True))
            out_ref[gb * BPB + j, 0] = xsum
            out_ref[gb * BPB + j, 1] = jnp.sum(jnp.maximum(colmin, 0.0))

    return pl.pallas_call(
        body,
        grid=(GB,),
        in_specs=[
            pl.BlockSpec((BPB, N, 3), lambda g: (g, 0, 0)),
            pl.BlockSpec((BPB, 3, N), lambda g: (g, 0, 0)),
        ],
        out_specs=pl.BlockSpec((B, 2), lambda g: (0, 0),
                               memory_space=pltpu.SMEM),
        out_shape=jax.ShapeDtypeStruct((B, 2), jnp.float32),
    )


_chamfer_tc = _build_tc(8, 2048, 1024, 2)


def kernel(pred_points, gt_points):
    B, N, _ = pred_points.shape
    pred = pred_points.astype(jnp.float32)
    gt = gt_points.astype(jnp.float32)
    g2T = jnp.swapaxes(-2.0 * gt, 1, 2)
    parts = _chamfer_tc(pred, g2T)
    return jnp.sum(parts) / (B * N)
